# Initial kernel scaffold; baseline (speedup 1.0000x reference)
import jax
import jax.numpy as jnp
from jax.experimental import pallas as pl


def kernel(his, loc, edges, vel, edge_attr, W_emb, b_emb, We1, be1, We2, be2, Wn1, bn1, Wn2, bn2, Wc1, bc1, Wc2, bc2, Wv1, bv1, Wv2, bv2):
    raise NotImplementedError("write your pallas kernel here")


# speedup vs baseline: 2.5928x; 2.5928x over previous
import jax
import jax.numpy as jnp
from jax import lax
from jax.experimental import pallas as pl
from jax.experimental.pallas import tpu as pltpu
from jax.experimental.pallas import tpu_sc as plsc

N = 10000
E = 320000
NPAD = 10240
EPAD = 327680
DUMMY = NPAD - 1
INNER = 7.0

NC, NS = 2, 16
NW = NC * NS
EW = EPAD // NW
BB = 128
NBLK = EW // BB
RPT = NPAD // NS

_mesh = plsc.VectorSubcoreMesh(
    core_axis_name="c", subcore_axis_name="s", num_cores=NC, num_subcores=NS)


def _silu(t):
    return t * jax.nn.sigmoid(t)


def _sc_gather_body(rowg, colg, ta, tb, ga_out, gb_out,
                    idxr, idxc, bufa, bufb, sem):
    c = lax.axis_index("c")
    s = lax.axis_index("s")
    base_w = (s * NC + c) * EW

    def step(k, carry):
        base = base_w + k * BB
        pltpu.sync_copy(rowg.at[pl.ds(base, BB)], idxr)
        pltpu.sync_copy(colg.at[pl.ds(base, BB)], idxc)
        ca = pltpu.async_copy(ta.at[idxr], bufa, sem)
        cb = pltpu.async_copy(tb.at[idxc], bufb, sem)
        ca.wait()
        cb.wait()
        pltpu.sync_copy(bufa, ga_out.at[pl.ds(base, BB)])
        pltpu.sync_copy(bufb, gb_out.at[pl.ds(base, BB)])
        return carry

    lax.fori_loop(0, NBLK, step, 0)


_gather_call = pl.kernel(
    _sc_gather_body,
    out_type=[
        jax.ShapeDtypeStruct((EPAD, 128), jnp.float32),
        jax.ShapeDtypeStruct((EPAD, 128), jnp.float32),
    ],
    mesh=_mesh,
    scratch_types=[
        pltpu.VMEM((BB,), jnp.int32),
        pltpu.VMEM((BB,), jnp.int32),
        pltpu.VMEM((BB, 128), jnp.float32),
        pltpu.VMEM((BB, 128), jnp.float32),
        pltpu.SemaphoreType.DMA,
    ],
)


def _sc_scatter_body(rowsc, mt, z128, part_out, idx, buf, sh, sem):
    c = lax.axis_index("c")
    s = lax.axis_index("s")
    r0 = s * RPT

    pltpu.sync_copy(z128, sh.at[pl.ds(r0, RPT)])
    plsc.subcore_barrier()

    base_w = (s * NC + c) * EW

    def step(k, carry):
        base = base_w + k * BB
        pltpu.sync_copy(rowsc.at[pl.ds(base, BB)], idx)
        pltpu.sync_copy(mt.at[pl.ds(base, BB)], buf)
        pltpu.sync_copy(buf, sh.at[idx], add=True)
        return carry

    lax.fori_loop(0, NBLK, step, 0)
    plsc.subcore_barrier()

    pltpu.sync_copy(sh.at[pl.ds(r0, RPT)], part_out.at[c, pl.ds(r0, RPT)])


_scatter_call = pl.kernel(
    _sc_scatter_body,
    out_type=jax.ShapeDtypeStruct((NC, NPAD, 128), jnp.float32),
    mesh=_mesh,
    scratch_types=[
        pltpu.VMEM((BB,), jnp.int32),
        pltpu.VMEM((BB, 128), jnp.float32),
        pltpu.VMEM_SHARED((NPAD, 128), jnp.float32),
        pltpu.SemaphoreType.DMA,
    ],
)


BE = 2048
BN = 1024


def _full(shape):
    return pl.BlockSpec(shape, lambda i: (0,) * len(shape))


def _edge_mlp_body(ga_ref, gb_ref, ea_ref, w1r_ref, w1e_ref, we2_ref, be2_ref,
                   wc1_ref, bc1_ref, wc2_ref, bc2_ref, mt_ref):
    ga = ga_ref[...]
    gb = gb_ref[...]
    pre = ga[:, 0:64] + gb[:, 0:64]
    cd = ga[:, 64:80] - gb[:, 64:80]
    radial = jnp.sum(cd * cd, axis=1, keepdims=True)
    ein = (pre + radial * w1r_ref[...]
           + jnp.dot(ea_ref[...], w1e_ref[...],
                     preferred_element_type=jnp.float32))
    a1 = _silu(ein)
    mm = _silu(jnp.dot(a1, we2_ref[...],
                       preferred_element_type=jnp.float32) + be2_ref[...])
    u = _silu(jnp.dot(mm, wc1_ref[...],
                      preferred_element_type=jnp.float32) + bc1_ref[...])
    sc = jnp.sum(u * wc2_ref[...], axis=1, keepdims=True) + bc2_ref[0, 0]
    lane = lax.broadcasted_iota(jnp.int32, cd.shape, 1)
    e3 = jnp.where(lane == 3, 1.0, 0.0).astype(jnp.float32)
    trans = cd * sc + e3
    zpad = jnp.zeros((trans.shape[0], 48), jnp.float32)
    mt_ref[...] = jnp.concatenate([mm, trans, zpad], axis=1)


_edge_mlp = pl.pallas_call(
    _edge_mlp_body,
    grid=(EPAD // BE,),
    in_specs=[
        pl.BlockSpec((BE, 128), lambda i: (i, 0)),
        pl.BlockSpec((BE, 128), lambda i: (i, 0)),
        pl.BlockSpec((BE, 8), lambda i: (i, 0)),
        _full((1, 64)), _full((8, 64)), _full((64, 64)), _full((1, 64)),
        _full((64, 64)), _full((1, 64)), _full((1, 64)), _full((1, 1)),
    ],
    out_specs=pl.BlockSpec((BE, 128), lambda i: (i, 0)),
    out_shape=jax.ShapeDtypeStruct((EPAD, 128), jnp.float32),
)


def _setup_body(his_ref, x_ref, v_ref, wemb_ref, bemb_ref, w1a_ref, b1_ref,
                w1b_ref, h0_ref, ta_ref, tb_ref, vh_ref):
    he = jnp.dot(his_ref[...], wemb_ref[...],
                 preferred_element_type=jnp.float32) + bemb_ref[...]
    h0_ref[...] = he
    p1 = jnp.dot(he, w1a_ref[...],
                 preferred_element_type=jnp.float32) + b1_ref[...]
    p2 = jnp.dot(he, w1b_ref[...], preferred_element_type=jnp.float32)
    x = x_ref[...]
    zpad = jnp.zeros((x.shape[0], 48), jnp.float32)
    ta_ref[...] = jnp.concatenate([p1, x, zpad], axis=1)
    tb_ref[...] = jnp.concatenate([p2, x, zpad], axis=1)
    v = v_ref[...]
    nv = jnp.sqrt(jnp.sum(v * v, axis=1, keepdims=True)) + 1.0
    vh_ref[...] = v / nv


_setup_call = pl.pallas_call(
    _setup_body,
    grid=(NPAD // BN,),
    in_specs=[
        pl.BlockSpec((BN, 128), lambda i: (i, 0)),
        pl.BlockSpec((BN, 16), lambda i: (i, 0)),
        pl.BlockSpec((BN, 16), lambda i: (i, 0)),
        _full((128, 64)), _full((1, 64)), _full((64, 64)), _full((1, 64)),
        _full((64, 64)),
    ],
    out_specs=[
        pl.BlockSpec((BN, 64), lambda i: (i, 0)),
        pl.BlockSpec((BN, 128), lambda i: (i, 0)),
        pl.BlockSpec((BN, 128), lambda i: (i, 0)),
        pl.BlockSpec((BN, 16), lambda i: (i, 0)),
    ],
    out_shape=[
        jax.ShapeDtypeStruct((NPAD, 64), jnp.float32),
        jax.ShapeDtypeStruct((NPAD, 128), jnp.float32),
        jax.ShapeDtypeStruct((NPAD, 128), jnp.float32),
        jax.ShapeDtypeStruct((NPAD, 16), jnp.float32),
    ],
)


def _node_body(h_ref, x_ref, v_ref, vh_ref, p0_ref, p1_ref,
               wn1a_ref, wn1b_ref, bn1_ref, wn2_ref, bn2_ref,
               wv1_ref, bv1_ref, wv2_ref, bv2_ref, w1a_ref, b1_ref, w1b_ref,
               h_out, x_out, v_out, ta_out, tb_out):
    hh = h_ref[...]
    mtsum = p0_ref[...] + p1_ref[...]
    aggh = mtsum[:, 0:64]
    act = mtsum[:, 64:80]
    cnt = jnp.maximum(act[:, 3:4], 1.0)
    lane = lax.broadcasted_iota(jnp.int32, act.shape, 1)
    accv = jnp.where(lane < 3, act, 0.0) / cnt
    sv = (jnp.sum(_silu(jnp.dot(hh, wv1_ref[...],
                                preferred_element_type=jnp.float32)
                        + bv1_ref[...]) * wv2_ref[...],
                  axis=1, keepdims=True) + bv2_ref[0, 0])
    vn = sv * vh_ref[...] + v_ref[...] + accv * (1.0 / INNER)
    xn = x_ref[...] + vn * (1.0 / INNER)
    s1 = _silu(jnp.dot(hh, wn1a_ref[...], preferred_element_type=jnp.float32)
               + jnp.dot(aggh, wn1b_ref[...],
                         preferred_element_type=jnp.float32) + bn1_ref[...])
    hn = 2.0 * hh + jnp.dot(s1, wn2_ref[...],
                            preferred_element_type=jnp.float32) + bn2_ref[...]
    h_out[...] = hn
    x_out[...] = xn
    v_out[...] = vn
    np1 = jnp.dot(hn, w1a_ref[...],
                  preferred_element_type=jnp.float32) + b1_ref[...]
    np2 = jnp.dot(hn, w1b_ref[...], preferred_element_type=jnp.float32)
    zpad = jnp.zeros((xn.shape[0], 48), jnp.float32)
    ta_out[...] = jnp.concatenate([np1, xn, zpad], axis=1)
    tb_out[...] = jnp.concatenate([np2, xn, zpad], axis=1)


_node_call = pl.pallas_call(
    _node_body,
    grid=(NPAD // BN,),
    in_specs=[
        pl.BlockSpec((BN, 64), lambda i: (i, 0)),
        pl.BlockSpec((BN, 16), lambda i: (i, 0)),
        pl.BlockSpec((BN, 16), lambda i: (i, 0)),
        pl.BlockSpec((BN, 16), lambda i: (i, 0)),
        pl.BlockSpec((BN, 128), lambda i: (i, 0)),
        pl.BlockSpec((BN, 128), lambda i: (i, 0)),
        _full((64, 64)), _full((64, 64)), _full((1, 64)),
        _full((64, 64)), _full((1, 64)),
        _full((64, 64)), _full((1, 64)), _full((1, 64)), _full((1, 1)),
        _full((64, 64)), _full((1, 64)), _full((64, 64)),
    ],
    out_specs=[
        pl.BlockSpec((BN, 64), lambda i: (i, 0)),
        pl.BlockSpec((BN, 16), lambda i: (i, 0)),
        pl.BlockSpec((BN, 16), lambda i: (i, 0)),
        pl.BlockSpec((BN, 128), lambda i: (i, 0)),
        pl.BlockSpec((BN, 128), lambda i: (i, 0)),
    ],
    out_shape=[
        jax.ShapeDtypeStruct((NPAD, 64), jnp.float32),
        jax.ShapeDtypeStruct((NPAD, 16), jnp.float32),
        jax.ShapeDtypeStruct((NPAD, 16), jnp.float32),
        jax.ShapeDtypeStruct((NPAD, 128), jnp.float32),
        jax.ShapeDtypeStruct((NPAD, 128), jnp.float32),
    ],
)


@jax.jit
def _run(his, loc, edges, vel, edge_attr, W_emb, b_emb, We1, be1, We2, be2,
         Wn1, bn1, Wn2, bn2, Wc1, bc1, Wc2, bc2, Wv1, bv1, Wv2, bv2):
    f32 = jnp.float32
    row, col = edges[0], edges[1]
    rowg = jnp.concatenate([row, jnp.zeros((EPAD - E,), jnp.int32)])
    colg = jnp.concatenate([col, jnp.zeros((EPAD - E,), jnp.int32)])
    rowsc = jnp.concatenate([row, jnp.full((EPAD - E,), DUMMY, jnp.int32)])

    his_p = jnp.pad(his, ((0, NPAD - N), (0, 0)))
    xpad0 = jnp.pad(loc, ((0, NPAD - N), (0, 13)))
    vpad0 = jnp.pad(vel, ((0, NPAD - N), (0, 13)))
    eap = jnp.pad(edge_attr, ((0, EPAD - E), (0, 4)))

    w1a = We1[0:64]
    w1b = We1[64:128]
    w1r = We1[128:129]
    w1e = jnp.pad(We1[129:133], ((0, 4), (0, 0)))
    r1 = lambda a: a.reshape(1, -1).astype(f32)
    be1r, be2r, bn1r, bn2r = r1(be1), r1(be2), r1(bn1), r1(bn2)
    bembr, bv1r, bc1r = r1(b_emb), r1(bv1), r1(bc1)
    wc2r, wv2r = r1(Wc2), r1(Wv2)
    bc2r, bv2r = bc2.reshape(1, 1), bv2.reshape(1, 1)

    h, ta, tb, velhat = _setup_call(
        his_p, xpad0, vpad0, W_emb, bembr, w1a, be1r, w1b)
    x, v = xpad0, vpad0

    z128 = jnp.zeros((RPT, 128), f32)

    for _ in range(4):
        ga, gb = _gather_call(rowg, colg, ta, tb)
        mt = _edge_mlp(ga, gb, eap, w1r, w1e, We2, be2r, Wc1, bc1r, wc2r, bc2r)
        part = _scatter_call(rowsc, mt, z128)
        h, x, v, ta, tb = _node_call(
            h, x, v, velhat, part[0], part[1],
            Wn1[0:64], Wn1[64:128], bn1r, Wn2, bn2r,
            Wv1, bv1r, wv2r, bv2r, w1a, be1r, w1b)

    return (x[:N, :3], h[:N], v[:N, :3])


def kernel(his, loc, edges, vel, edge_attr, W_emb, b_emb, We1, be1, We2, be2,
           Wn1, bn1, Wn2, bn2, Wc1, bc1, Wc2, bc2, Wv1, bv1, Wv2, bv2):
    return _run(his, loc, edges, vel, edge_attr, W_emb, b_emb, We1, be1,
                We2, be2, Wn1, bn1, Wn2, bn2, Wc1, bc1, Wc2, bc2,
                Wv1, bv1, Wv2, bv2)


# speedup vs baseline: 3.4538x; 1.3321x over previous
import jax
import jax.numpy as jnp
from jax import lax
from jax.experimental import pallas as pl
from jax.experimental.pallas import tpu as pltpu
from jax.experimental.pallas import tpu_sc as plsc

N = 10000
E = 320000
NPAD = 10240
EPAD = 327680
DUMMY = NPAD - 1
INNER = 7.0

NC, NS = 2, 16
NW = NC * NS
EW = EPAD // NW
BB = 128
NBLK = EW // BB
RPT = NPAD // NS

_mesh = plsc.VectorSubcoreMesh(
    core_axis_name="c", subcore_axis_name="s", num_cores=NC, num_subcores=NS)


def _silu(t):
    return t * jax.nn.sigmoid(t)


def _sc_gather_body(rowg, colg, ta, tb, ga_out, gb_out,
                    idxr, idxc, bufa0, bufb0, bufa1, bufb1,
                    sga0, sgb0, sga1, sgb1, swa0, swb0, swa1, swb1):
    c = lax.axis_index("c")
    s = lax.axis_index("s")
    w = s * NC + c
    base_w = w * EW

    pltpu.sync_copy(rowg.at[w], idxr)
    pltpu.sync_copy(colg.at[w], idxc)

    bufa = [bufa0, bufa1]
    bufb = [bufb0, bufb1]
    sga = [sga0, sga1]
    sgb = [sgb0, sgb1]
    swa = [swa0, swa1]
    swb = [swb0, swb1]

    def issue_gather(k, b):
        pltpu.async_copy(ta.at[idxr.at[k]], bufa[b], sga[b])
        pltpu.async_copy(tb.at[idxc.at[k]], bufb[b], sgb[b])

    issue_gather(0, 0)
    issue_gather(1, 1)

    def outer(g, carry):
        for b in range(2):
            k = 2 * g + b
            base = base_w + k * BB
            pltpu.make_async_copy(ta.at[idxr.at[k]], bufa[b], sga[b]).wait()
            pltpu.make_async_copy(tb.at[idxc.at[k]], bufb[b], sgb[b]).wait()
            pltpu.async_copy(bufa[b], ga_out.at[pl.ds(base, BB)], swa[b])
            pltpu.async_copy(bufb[b], gb_out.at[pl.ds(base, BB)], swb[b])
            pltpu.make_async_copy(bufa[b], ga_out.at[pl.ds(base, BB)],
                                  swa[b]).wait()
            pltpu.make_async_copy(bufb[b], gb_out.at[pl.ds(base, BB)],
                                  swb[b]).wait()

            @pl.when(k + 2 < NBLK)
            def _():
                pltpu.async_copy(ta.at[idxr.at[k + 2]], bufa[b], sga[b])
                pltpu.async_copy(tb.at[idxc.at[k + 2]], bufb[b], sgb[b])
        return carry

    lax.fori_loop(0, NBLK // 2, outer, 0)


_gather_call = pl.kernel(
    _sc_gather_body,
    out_type=[
        jax.ShapeDtypeStruct((EPAD, 128), jnp.float32),
        jax.ShapeDtypeStruct((EPAD, 128), jnp.float32),
    ],
    mesh=_mesh,
    scratch_types=[
        pltpu.VMEM((NBLK, BB), jnp.int32),
        pltpu.VMEM((NBLK, BB), jnp.int32),
        pltpu.VMEM((BB, 128), jnp.float32),
        pltpu.VMEM((BB, 128), jnp.float32),
        pltpu.VMEM((BB, 128), jnp.float32),
        pltpu.VMEM((BB, 128), jnp.float32),
        pltpu.SemaphoreType.DMA,
        pltpu.SemaphoreType.DMA,
        pltpu.SemaphoreType.DMA,
        pltpu.SemaphoreType.DMA,
        pltpu.SemaphoreType.DMA,
        pltpu.SemaphoreType.DMA,
        pltpu.SemaphoreType.DMA,
        pltpu.SemaphoreType.DMA,
    ],
)


def _sc_scatter_body(rowsc, mt, zrow, part_out, idx, buf0, buf1, sh,
                     sem0, sem1):
    c = lax.axis_index("c")
    s = lax.axis_index("s")
    w = s * NC + c
    r0 = s * RPT
    base_w = w * EW

    pltpu.sync_copy(rowsc.at[w], idx)

    pltpu.sync_copy(zrow, buf0)
    for r in range(RPT // BB):
        pltpu.sync_copy(buf0, sh.at[pl.ds(r0 + r * BB, BB)])
    plsc.subcore_barrier()

    buf = [buf0, buf1]
    sem = [sem0, sem1]

    def issue_load(k, b):
        pltpu.async_copy(mt.at[pl.ds(base_w + k * BB, BB)], buf[b], sem[b])

    issue_load(0, 0)
    issue_load(1, 1)

    def outer(g, carry):
        for b in range(2):
            k = 2 * g + b
            pltpu.make_async_copy(mt.at[pl.ds(base_w + k * BB, BB)],
                                  buf[b], sem[b]).wait()
            pltpu.sync_copy(buf[b], sh.at[idx.at[k]], add=True)

            @pl.when(k + 2 < NBLK)
            def _():
                issue_load(k + 2, b)
        return carry

    lax.fori_loop(0, NBLK // 2, outer, 0)
    plsc.subcore_barrier()

    pltpu.sync_copy(sh.at[pl.ds(r0, RPT)], part_out.at[c, pl.ds(r0, RPT)])


_scatter_call = pl.kernel(
    _sc_scatter_body,
    out_type=jax.ShapeDtypeStruct((NC, NPAD, 128), jnp.float32),
    mesh=_mesh,
    scratch_types=[
        pltpu.VMEM((NBLK, BB), jnp.int32),
        pltpu.VMEM((BB, 128), jnp.float32),
        pltpu.VMEM((BB, 128), jnp.float32),
        pltpu.VMEM_SHARED((NPAD, 128), jnp.float32),
        pltpu.SemaphoreType.DMA,
        pltpu.SemaphoreType.DMA,
    ],
)


BE = 2048
BN = 1024


def _full(shape):
    return pl.BlockSpec(shape, lambda i: (0,) * len(shape))


def _edge_mlp_body(ga_ref, gb_ref, ea_ref, w1r_ref, w1e_ref, we2_ref, be2_ref,
                   wc1_ref, bc1_ref, wc2_ref, bc2_ref, mt_ref):
    ga = ga_ref[...]
    gb = gb_ref[...]
    pre = ga[:, 0:64] + gb[:, 0:64]
    cd = ga[:, 64:80] - gb[:, 64:80]
    radial = jnp.sum(cd * cd, axis=1, keepdims=True)
    ein = (pre + radial * w1r_ref[...]
           + jnp.dot(ea_ref[...], w1e_ref[...],
                     preferred_element_type=jnp.float32))
    a1 = _silu(ein)
    mm = _silu(jnp.dot(a1, we2_ref[...],
                       preferred_element_type=jnp.float32) + be2_ref[...])
    u = _silu(jnp.dot(mm, wc1_ref[...],
                      preferred_element_type=jnp.float32) + bc1_ref[...])
    sc = jnp.sum(u * wc2_ref[...], axis=1, keepdims=True) + bc2_ref[0, 0]
    lane = lax.broadcasted_iota(jnp.int32, cd.shape, 1)
    e3 = jnp.where(lane == 3, 1.0, 0.0).astype(jnp.float32)
    trans = cd * sc + e3
    zpad = jnp.zeros((trans.shape[0], 48), jnp.float32)
    mt_ref[...] = jnp.concatenate([mm, trans, zpad], axis=1)


_edge_mlp = pl.pallas_call(
    _edge_mlp_body,
    grid=(EPAD // BE,),
    in_specs=[
        pl.BlockSpec((BE, 128), lambda i: (i, 0)),
        pl.BlockSpec((BE, 128), lambda i: (i, 0)),
        pl.BlockSpec((BE, 8), lambda i: (i, 0)),
        _full((1, 64)), _full((8, 64)), _full((64, 64)), _full((1, 64)),
        _full((64, 64)), _full((1, 64)), _full((1, 64)), _full((1, 1)),
    ],
    out_specs=pl.BlockSpec((BE, 128), lambda i: (i, 0)),
    out_shape=jax.ShapeDtypeStruct((EPAD, 128), jnp.float32),
)


def _setup_body(his_ref, x_ref, v_ref, wemb_ref, bemb_ref, w1a_ref, b1_ref,
                w1b_ref, h0_ref, ta_ref, tb_ref, vh_ref):
    he = jnp.dot(his_ref[...], wemb_ref[...],
                 preferred_element_type=jnp.float32) + bemb_ref[...]
    h0_ref[...] = he
    p1 = jnp.dot(he, w1a_ref[...],
                 preferred_element_type=jnp.float32) + b1_ref[...]
    p2 = jnp.dot(he, w1b_ref[...], preferred_element_type=jnp.float32)
    x = x_ref[...]
    zpad = jnp.zeros((x.shape[0], 48), jnp.float32)
    ta_ref[...] = jnp.concatenate([p1, x, zpad], axis=1)
    tb_ref[...] = jnp.concatenate([p2, x, zpad], axis=1)
    v = v_ref[...]
    nv = jnp.sqrt(jnp.sum(v * v, axis=1, keepdims=True)) + 1.0
    vh_ref[...] = v / nv


_setup_call = pl.pallas_call(
    _setup_body,
    grid=(NPAD // BN,),
    in_specs=[
        pl.BlockSpec((BN, 128), lambda i: (i, 0)),
        pl.BlockSpec((BN, 16), lambda i: (i, 0)),
        pl.BlockSpec((BN, 16), lambda i: (i, 0)),
        _full((128, 64)), _full((1, 64)), _full((64, 64)), _full((1, 64)),
        _full((64, 64)),
    ],
    out_specs=[
        pl.BlockSpec((BN, 64), lambda i: (i, 0)),
        pl.BlockSpec((BN, 128), lambda i: (i, 0)),
        pl.BlockSpec((BN, 128), lambda i: (i, 0)),
        pl.BlockSpec((BN, 16), lambda i: (i, 0)),
    ],
    out_shape=[
        jax.ShapeDtypeStruct((NPAD, 64), jnp.float32),
        jax.ShapeDtypeStruct((NPAD, 128), jnp.float32),
        jax.ShapeDtypeStruct((NPAD, 128), jnp.float32),
        jax.ShapeDtypeStruct((NPAD, 16), jnp.float32),
    ],
)


def _node_body(h_ref, x_ref, v_ref, vh_ref, p0_ref, p1_ref,
               wn1a_ref, wn1b_ref, bn1_ref, wn2_ref, bn2_ref,
               wv1_ref, bv1_ref, wv2_ref, bv2_ref, w1a_ref, b1_ref, w1b_ref,
               h_out, x_out, v_out, ta_out, tb_out):
    hh = h_ref[...]
    mtsum = p0_ref[...] + p1_ref[...]
    aggh = mtsum[:, 0:64]
    act = mtsum[:, 64:80]
    cnt = jnp.maximum(act[:, 3:4], 1.0)
    lane = lax.broadcasted_iota(jnp.int32, act.shape, 1)
    accv = jnp.where(lane < 3, act, 0.0) / cnt
    sv = (jnp.sum(_silu(jnp.dot(hh, wv1_ref[...],
                                preferred_element_type=jnp.float32)
                        + bv1_ref[...]) * wv2_ref[...],
                  axis=1, keepdims=True) + bv2_ref[0, 0])
    vn = sv * vh_ref[...] + v_ref[...] + accv * (1.0 / INNER)
    xn = x_ref[...] + vn * (1.0 / INNER)
    s1 = _silu(jnp.dot(hh, wn1a_ref[...], preferred_element_type=jnp.float32)
               + jnp.dot(aggh, wn1b_ref[...],
                         preferred_element_type=jnp.float32) + bn1_ref[...])
    hn = 2.0 * hh + jnp.dot(s1, wn2_ref[...],
                            preferred_element_type=jnp.float32) + bn2_ref[...]
    h_out[...] = hn
    x_out[...] = xn
    v_out[...] = vn
    np1 = jnp.dot(hn, w1a_ref[...],
                  preferred_element_type=jnp.float32) + b1_ref[...]
    np2 = jnp.dot(hn, w1b_ref[...], preferred_element_type=jnp.float32)
    zpad = jnp.zeros((xn.shape[0], 48), jnp.float32)
    ta_out[...] = jnp.concatenate([np1, xn, zpad], axis=1)
    tb_out[...] = jnp.concatenate([np2, xn, zpad], axis=1)


_node_call = pl.pallas_call(
    _node_body,
    grid=(NPAD // BN,),
    in_specs=[
        pl.BlockSpec((BN, 64), lambda i: (i, 0)),
        pl.BlockSpec((BN, 16), lambda i: (i, 0)),
        pl.BlockSpec((BN, 16), lambda i: (i, 0)),
        pl.BlockSpec((BN, 16), lambda i: (i, 0)),
        pl.BlockSpec((BN, 128), lambda i: (i, 0)),
        pl.BlockSpec((BN, 128), lambda i: (i, 0)),
        _full((64, 64)), _full((64, 64)), _full((1, 64)),
        _full((64, 64)), _full((1, 64)),
        _full((64, 64)), _full((1, 64)), _full((1, 64)), _full((1, 1)),
        _full((64, 64)), _full((1, 64)), _full((64, 64)),
    ],
    out_specs=[
        pl.BlockSpec((BN, 64), lambda i: (i, 0)),
        pl.BlockSpec((BN, 16), lambda i: (i, 0)),
        pl.BlockSpec((BN, 16), lambda i: (i, 0)),
        pl.BlockSpec((BN, 128), lambda i: (i, 0)),
        pl.BlockSpec((BN, 128), lambda i: (i, 0)),
    ],
    out_shape=[
        jax.ShapeDtypeStruct((NPAD, 64), jnp.float32),
        jax.ShapeDtypeStruct((NPAD, 16), jnp.float32),
        jax.ShapeDtypeStruct((NPAD, 16), jnp.float32),
        jax.ShapeDtypeStruct((NPAD, 128), jnp.float32),
        jax.ShapeDtypeStruct((NPAD, 128), jnp.float32),
    ],
)


@jax.jit
def _run(his, loc, edges, vel, edge_attr, W_emb, b_emb, We1, be1, We2, be2,
         Wn1, bn1, Wn2, bn2, Wc1, bc1, Wc2, bc2, Wv1, bv1, Wv2, bv2):
    f32 = jnp.float32
    row, col = edges[0], edges[1]
    rowg = jnp.concatenate(
        [row, jnp.zeros((EPAD - E,), jnp.int32)]).reshape(NW, NBLK, BB)
    colg = jnp.concatenate(
        [col, jnp.zeros((EPAD - E,), jnp.int32)]).reshape(NW, NBLK, BB)
    rowsc = jnp.concatenate(
        [row, jnp.full((EPAD - E,), DUMMY, jnp.int32)]).reshape(NW, NBLK, BB)

    his_p = jnp.pad(his, ((0, NPAD - N), (0, 0)))
    xpad0 = jnp.pad(loc, ((0, NPAD - N), (0, 13)))
    vpad0 = jnp.pad(vel, ((0, NPAD - N), (0, 13)))
    eap = jnp.pad(edge_attr, ((0, EPAD - E), (0, 4)))

    w1a = We1[0:64]
    w1b = We1[64:128]
    w1r = We1[128:129]
    w1e = jnp.pad(We1[129:133], ((0, 4), (0, 0)))
    r1 = lambda a: a.reshape(1, -1).astype(f32)
    be1r, be2r, bn1r, bn2r = r1(be1), r1(be2), r1(bn1), r1(bn2)
    bembr, bv1r, bc1r = r1(b_emb), r1(bv1), r1(bc1)
    wc2r, wv2r = r1(Wc2), r1(Wv2)
    bc2r, bv2r = bc2.reshape(1, 1), bv2.reshape(1, 1)

    h, ta, tb, velhat = _setup_call(
        his_p, xpad0, vpad0, W_emb, bembr, w1a, be1r, w1b)
    x, v = xpad0, vpad0

    zrow = jnp.zeros((BB, 128), f32)

    for _ in range(4):
        ga, gb = _gather_call(rowg, colg, ta, tb)
        mt = _edge_mlp(ga, gb, eap, w1r, w1e, We2, be2r, Wc1, bc1r, wc2r, bc2r)
        part = _scatter_call(rowsc, mt, zrow)
        h, x, v, ta, tb = _node_call(
            h, x, v, velhat, part[0], part[1],
            Wn1[0:64], Wn1[64:128], bn1r, Wn2, bn2r,
            Wv1, bv1r, wv2r, bv2r, w1a, be1r, w1b)

    return (x[:N, :3], h[:N], v[:N, :3])


def kernel(his, loc, edges, vel, edge_attr, W_emb, b_emb, We1, be1, We2, be2,
           Wn1, bn1, Wn2, bn2, Wc1, bc1, Wc2, bc2, Wv1, bv1, Wv2, bv2):
    return _run(his, loc, edges, vel, edge_attr, W_emb, b_emb, We1, be1,
                We2, be2, Wn1, bn1, Wn2, bn2, Wc1, bc1, Wc2, bc2,
                Wv1, bv1, Wv2, bv2)


# speedup vs baseline: 3.4594x; 1.0016x over previous
import jax
import jax.numpy as jnp
from jax import lax
from jax.experimental import pallas as pl
from jax.experimental.pallas import tpu as pltpu
from jax.experimental.pallas import tpu_sc as plsc

N = 10000
E = 320000
NPAD = 10240
EPAD = 327680
DUMMY = NPAD - 1
INNER = 7.0

NC, NS = 2, 16
NW = NC * NS
EW = EPAD // NW
BB = 128
NBLK = EW // BB
RPT = NPAD // NS

_mesh = plsc.VectorSubcoreMesh(
    core_axis_name="c", subcore_axis_name="s", num_cores=NC, num_subcores=NS)


def _silu(t):
    return t * jax.nn.sigmoid(t)


NBUF = 3
GMAIN = (NBLK // NBUF) * NBUF


def _sc_gather_body(rowg, colg, ta, tb, ga_out, gb_out,
                    idxr, idxc,
                    bufa0, bufb0, bufa1, bufb1, bufa2, bufb2,
                    sga0, sgb0, sga1, sgb1, sga2, sgb2,
                    swa0, swb0, swa1, swb1, swa2, swb2):
    c = lax.axis_index("c")
    s = lax.axis_index("s")
    w = s * NC + c
    base_w = w * EW

    pltpu.sync_copy(rowg.at[w], idxr)
    pltpu.sync_copy(colg.at[w], idxc)

    bufa = [bufa0, bufa1, bufa2]
    bufb = [bufb0, bufb1, bufb2]
    sga = [sga0, sga1, sga2]
    sgb = [sgb0, sgb1, sgb2]
    swa = [swa0, swa1, swa2]
    swb = [swb0, swb1, swb2]

    def issue_gather(k, b):
        pltpu.async_copy(ta.at[idxr.at[k]], bufa[b], sga[b])
        pltpu.async_copy(tb.at[idxc.at[k]], bufb[b], sgb[b])

    def consume(k, b, reissue):
        base = base_w + k * BB
        pltpu.make_async_copy(ta.at[idxr.at[k]], bufa[b], sga[b]).wait()
        pltpu.make_async_copy(tb.at[idxc.at[k]], bufb[b], sgb[b]).wait()
        pltpu.async_copy(bufa[b], ga_out.at[pl.ds(base, BB)], swa[b])
        pltpu.async_copy(bufb[b], gb_out.at[pl.ds(base, BB)], swb[b])
        pltpu.make_async_copy(bufa[b], ga_out.at[pl.ds(base, BB)],
                              swa[b]).wait()
        pltpu.make_async_copy(bufb[b], gb_out.at[pl.ds(base, BB)],
                              swb[b]).wait()
        if reissue:
            @pl.when(k + NBUF < NBLK)
            def _():
                pltpu.async_copy(ta.at[idxr.at[k + NBUF]], bufa[b], sga[b])
                pltpu.async_copy(tb.at[idxc.at[k + NBUF]], bufb[b], sgb[b])

    for b in range(NBUF):
        issue_gather(b, b)

    def outer(g, carry):
        for b in range(NBUF):
            consume(NBUF * g + b, b, True)
        return carry

    lax.fori_loop(0, GMAIN // NBUF, outer, 0)
    for k in range(GMAIN, NBLK):
        consume(k, k % NBUF, False)


_gather_call = pl.kernel(
    _sc_gather_body,
    out_type=[
        jax.ShapeDtypeStruct((EPAD, 128), jnp.float32),
        jax.ShapeDtypeStruct((EPAD, 128), jnp.float32),
    ],
    mesh=_mesh,
    scratch_types=(
        [pltpu.VMEM((NBLK, BB), jnp.int32)] * 2
        + [pltpu.VMEM((BB, 128), jnp.float32)] * (2 * NBUF)
        + [pltpu.SemaphoreType.DMA] * (4 * NBUF)
    ),
)


SNBUF = 2
SMAIN = (NBLK // SNBUF) * SNBUF


def _sc_scatter_body(rowsc, mt, zrow, part_out, idx, buf0, buf1, sh,
                     sem0, sem1):
    c = lax.axis_index("c")
    s = lax.axis_index("s")
    w = s * NC + c
    r0 = s * RPT
    base_w = w * EW

    pltpu.sync_copy(rowsc.at[w], idx)

    pltpu.sync_copy(zrow, buf0)
    for r in range(RPT // BB):
        pltpu.sync_copy(buf0, sh.at[pl.ds(r0 + r * BB, BB)])
    plsc.subcore_barrier()

    buf = [buf0, buf1]
    sem = [sem0, sem1]

    def issue_load(k, b):
        pltpu.async_copy(mt.at[pl.ds(base_w + k * BB, BB)], buf[b], sem[b])

    def consume(k, b, reissue):
        pltpu.make_async_copy(mt.at[pl.ds(base_w + k * BB, BB)],
                              buf[b], sem[b]).wait()
        pltpu.sync_copy(buf[b], sh.at[idx.at[k]], add=True)
        if reissue:
            @pl.when(k + SNBUF < NBLK)
            def _():
                issue_load(k + SNBUF, b)

    for b in range(SNBUF):
        issue_load(b, b)

    def outer(g, carry):
        for b in range(SNBUF):
            consume(SNBUF * g + b, b, True)
        return carry

    lax.fori_loop(0, SMAIN // SNBUF, outer, 0)
    for k in range(SMAIN, NBLK):
        consume(k, k % SNBUF, False)
    plsc.subcore_barrier()

    pltpu.sync_copy(sh.at[pl.ds(r0, RPT)], part_out.at[c, pl.ds(r0, RPT)])


_scatter_call = pl.kernel(
    _sc_scatter_body,
    out_type=jax.ShapeDtypeStruct((NC, NPAD, 128), jnp.float32),
    mesh=_mesh,
    scratch_types=(
        [pltpu.VMEM((NBLK, BB), jnp.int32)]
        + [pltpu.VMEM((BB, 128), jnp.float32)] * SNBUF
        + [pltpu.VMEM_SHARED((NPAD, 128), jnp.float32)]
        + [pltpu.SemaphoreType.DMA] * SNBUF
    ),
)


BE = 2048
BN = 1024


def _full(shape):
    return pl.BlockSpec(shape, lambda i: (0,) * len(shape))


def _edge_mlp_body(ga_ref, gb_ref, ea_ref, w1r_ref, w1e_ref, we2_ref, be2_ref,
                   wc1_ref, bc1_ref, wc2_ref, bc2_ref, mt_ref):
    ga = ga_ref[...]
    gb = gb_ref[...]
    pre = ga[:, 0:64] + gb[:, 0:64]
    cd = ga[:, 64:80] - gb[:, 64:80]
    radial = jnp.sum(cd * cd, axis=1, keepdims=True)
    ein = (pre + radial * w1r_ref[...]
           + jnp.dot(ea_ref[...], w1e_ref[...],
                     preferred_element_type=jnp.float32))
    a1 = _silu(ein)
    mm = _silu(jnp.dot(a1, we2_ref[...],
                       preferred_element_type=jnp.float32) + be2_ref[...])
    u = _silu(jnp.dot(mm, wc1_ref[...],
                      preferred_element_type=jnp.float32) + bc1_ref[...])
    sc = jnp.sum(u * wc2_ref[...], axis=1, keepdims=True) + bc2_ref[0, 0]
    lane = lax.broadcasted_iota(jnp.int32, cd.shape, 1)
    e3 = jnp.where(lane == 3, 1.0, 0.0).astype(jnp.float32)
    trans = cd * sc + e3
    zpad = jnp.zeros((trans.shape[0], 48), jnp.float32)
    mt_ref[...] = jnp.concatenate([mm, trans, zpad], axis=1)


_edge_mlp = pl.pallas_call(
    _edge_mlp_body,
    grid=(EPAD // BE,),
    in_specs=[
        pl.BlockSpec((BE, 128), lambda i: (i, 0)),
        pl.BlockSpec((BE, 128), lambda i: (i, 0)),
        pl.BlockSpec((BE, 8), lambda i: (i, 0)),
        _full((1, 64)), _full((8, 64)), _full((64, 64)), _full((1, 64)),
        _full((64, 64)), _full((1, 64)), _full((1, 64)), _full((1, 1)),
    ],
    out_specs=pl.BlockSpec((BE, 128), lambda i: (i, 0)),
    out_shape=jax.ShapeDtypeStruct((EPAD, 128), jnp.float32),
)


def _setup_body(his_ref, x_ref, v_ref, wemb_ref, bemb_ref, w1a_ref, b1_ref,
                w1b_ref, h0_ref, ta_ref, tb_ref, vh_ref):
    he = jnp.dot(his_ref[...], wemb_ref[...],
                 preferred_element_type=jnp.float32) + bemb_ref[...]
    h0_ref[...] = he
    p1 = jnp.dot(he, w1a_ref[...],
                 preferred_element_type=jnp.float32) + b1_ref[...]
    p2 = jnp.dot(he, w1b_ref[...], preferred_element_type=jnp.float32)
    x = x_ref[...]
    zpad = jnp.zeros((x.shape[0], 48), jnp.float32)
    ta_ref[...] = jnp.concatenate([p1, x, zpad], axis=1)
    tb_ref[...] = jnp.concatenate([p2, x, zpad], axis=1)
    v = v_ref[...]
    nv = jnp.sqrt(jnp.sum(v * v, axis=1, keepdims=True)) + 1.0
    vh_ref[...] = v / nv


_setup_call = pl.pallas_call(
    _setup_body,
    grid=(NPAD // BN,),
    in_specs=[
        pl.BlockSpec((BN, 128), lambda i: (i, 0)),
        pl.BlockSpec((BN, 16), lambda i: (i, 0)),
        pl.BlockSpec((BN, 16), lambda i: (i, 0)),
        _full((128, 64)), _full((1, 64)), _full((64, 64)), _full((1, 64)),
        _full((64, 64)),
    ],
    out_specs=[
        pl.BlockSpec((BN, 64), lambda i: (i, 0)),
        pl.BlockSpec((BN, 128), lambda i: (i, 0)),
        pl.BlockSpec((BN, 128), lambda i: (i, 0)),
        pl.BlockSpec((BN, 16), lambda i: (i, 0)),
    ],
    out_shape=[
        jax.ShapeDtypeStruct((NPAD, 64), jnp.float32),
        jax.ShapeDtypeStruct((NPAD, 128), jnp.float32),
        jax.ShapeDtypeStruct((NPAD, 128), jnp.float32),
        jax.ShapeDtypeStruct((NPAD, 16), jnp.float32),
    ],
)


def _node_body(h_ref, x_ref, v_ref, vh_ref, p0_ref, p1_ref,
               wn1a_ref, wn1b_ref, bn1_ref, wn2_ref, bn2_ref,
               wv1_ref, bv1_ref, wv2_ref, bv2_ref, w1a_ref, b1_ref, w1b_ref,
               h_out, x_out, v_out, ta_out, tb_out):
    hh = h_ref[...]
    mtsum = p0_ref[...] + p1_ref[...]
    aggh = mtsum[:, 0:64]
    act = mtsum[:, 64:80]
    cnt = jnp.maximum(act[:, 3:4], 1.0)
    lane = lax.broadcasted_iota(jnp.int32, act.shape, 1)
    accv = jnp.where(lane < 3, act, 0.0) / cnt
    sv = (jnp.sum(_silu(jnp.dot(hh, wv1_ref[...],
                                preferred_element_type=jnp.float32)
                        + bv1_ref[...]) * wv2_ref[...],
                  axis=1, keepdims=True) + bv2_ref[0, 0])
    vn = sv * vh_ref[...] + v_ref[...] + accv * (1.0 / INNER)
    xn = x_ref[...] + vn * (1.0 / INNER)
    s1 = _silu(jnp.dot(hh, wn1a_ref[...], preferred_element_type=jnp.float32)
               + jnp.dot(aggh, wn1b_ref[...],
                         preferred_element_type=jnp.float32) + bn1_ref[...])
    hn = 2.0 * hh + jnp.dot(s1, wn2_ref[...],
                            preferred_element_type=jnp.float32) + bn2_ref[...]
    h_out[...] = hn
    x_out[...] = xn
    v_out[...] = vn
    np1 = jnp.dot(hn, w1a_ref[...],
                  preferred_element_type=jnp.float32) + b1_ref[...]
    np2 = jnp.dot(hn, w1b_ref[...], preferred_element_type=jnp.float32)
    zpad = jnp.zeros((xn.shape[0], 48), jnp.float32)
    ta_out[...] = jnp.concatenate([np1, xn, zpad], axis=1)
    tb_out[...] = jnp.concatenate([np2, xn, zpad], axis=1)


_node_call = pl.pallas_call(
    _node_body,
    grid=(NPAD // BN,),
    in_specs=[
        pl.BlockSpec((BN, 64), lambda i: (i, 0)),
        pl.BlockSpec((BN, 16), lambda i: (i, 0)),
        pl.BlockSpec((BN, 16), lambda i: (i, 0)),
        pl.BlockSpec((BN, 16), lambda i: (i, 0)),
        pl.BlockSpec((BN, 128), lambda i: (i, 0)),
        pl.BlockSpec((BN, 128), lambda i: (i, 0)),
        _full((64, 64)), _full((64, 64)), _full((1, 64)),
        _full((64, 64)), _full((1, 64)),
        _full((64, 64)), _full((1, 64)), _full((1, 64)), _full((1, 1)),
        _full((64, 64)), _full((1, 64)), _full((64, 64)),
    ],
    out_specs=[
        pl.BlockSpec((BN, 64), lambda i: (i, 0)),
        pl.BlockSpec((BN, 16), lambda i: (i, 0)),
        pl.BlockSpec((BN, 16), lambda i: (i, 0)),
        pl.BlockSpec((BN, 128), lambda i: (i, 0)),
        pl.BlockSpec((BN, 128), lambda i: (i, 0)),
    ],
    out_shape=[
        jax.ShapeDtypeStruct((NPAD, 64), jnp.float32),
        jax.ShapeDtypeStruct((NPAD, 16), jnp.float32),
        jax.ShapeDtypeStruct((NPAD, 16), jnp.float32),
        jax.ShapeDtypeStruct((NPAD, 128), jnp.float32),
        jax.ShapeDtypeStruct((NPAD, 128), jnp.float32),
    ],
)


@jax.jit
def _run(his, loc, edges, vel, edge_attr, W_emb, b_emb, We1, be1, We2, be2,
         Wn1, bn1, Wn2, bn2, Wc1, bc1, Wc2, bc2, Wv1, bv1, Wv2, bv2):
    f32 = jnp.float32
    row, col = edges[0], edges[1]
    rowg = jnp.concatenate(
        [row, jnp.zeros((EPAD - E,), jnp.int32)]).reshape(NW, NBLK, BB)
    colg = jnp.concatenate(
        [col, jnp.zeros((EPAD - E,), jnp.int32)]).reshape(NW, NBLK, BB)
    rowsc = jnp.concatenate(
        [row, jnp.full((EPAD - E,), DUMMY, jnp.int32)]).reshape(NW, NBLK, BB)

    his_p = jnp.pad(his, ((0, NPAD - N), (0, 0)))
    xpad0 = jnp.pad(loc, ((0, NPAD - N), (0, 13)))
    vpad0 = jnp.pad(vel, ((0, NPAD - N), (0, 13)))
    eap = jnp.pad(edge_attr, ((0, EPAD - E), (0, 4)))

    w1a = We1[0:64]
    w1b = We1[64:128]
    w1r = We1[128:129]
    w1e = jnp.pad(We1[129:133], ((0, 4), (0, 0)))
    r1 = lambda a: a.reshape(1, -1).astype(f32)
    be1r, be2r, bn1r, bn2r = r1(be1), r1(be2), r1(bn1), r1(bn2)
    bembr, bv1r, bc1r = r1(b_emb), r1(bv1), r1(bc1)
    wc2r, wv2r = r1(Wc2), r1(Wv2)
    bc2r, bv2r = bc2.reshape(1, 1), bv2.reshape(1, 1)

    h, ta, tb, velhat = _setup_call(
        his_p, xpad0, vpad0, W_emb, bembr, w1a, be1r, w1b)
    x, v = xpad0, vpad0

    zrow = jnp.zeros((BB, 128), f32)

    for _ in range(4):
        ga, gb = _gather_call(rowg, colg, ta, tb)
        mt = _edge_mlp(ga, gb, eap, w1r, w1e, We2, be2r, Wc1, bc1r, wc2r, bc2r)
        part = _scatter_call(rowsc, mt, zrow)
        h, x, v, ta, tb = _node_call(
            h, x, v, velhat, part[0], part[1],
            Wn1[0:64], Wn1[64:128], bn1r, Wn2, bn2r,
            Wv1, bv1r, wv2r, bv2r, w1a, be1r, w1b)

    return (x[:N, :3], h[:N], v[:N, :3])


def kernel(his, loc, edges, vel, edge_attr, W_emb, b_emb, We1, be1, We2, be2,
           Wn1, bn1, Wn2, bn2, Wc1, bc1, Wc2, bc2, Wv1, bv1, Wv2, bv2):
    return _run(his, loc, edges, vel, edge_attr, W_emb, b_emb, We1, be1,
                We2, be2, Wn1, bn1, Wn2, bn2, Wc1, bc1, Wc2, bc2,
                Wv1, bv1, Wv2, bv2)
